# Initial kernel scaffold; baseline (speedup 1.0000x reference)
import jax
import jax.numpy as jnp
from jax.experimental import pallas as pl


def kernel(x, adj, W1, a1_src, a1_dst, W2, a2_src, a2_dst):
    raise NotImplementedError("write your pallas kernel here")


# speedup vs baseline: 16.3767x; 16.3767x over previous
import functools

import jax
import jax.numpy as jnp
from jax import lax
from jax.experimental import pallas as pl
from jax.experimental.pallas import tpu as pltpu
from jax.experimental.pallas import tpu_sc as plsc

_N = 10000
_E = 320000
_SLOPE = 0.2

_NC = 2
_NS = 16
_NT = _NC * _NS
_B = 80
_EPT = _E // _NT
_CHUNKS = _EPT // _B
_RPT = _N // _NS
_ZR = 125

_BLK = 1000
_HIGH = jax.lax.Precision.HIGHEST


def _sc_aggregate(D):
    mesh = plsc.VectorSubcoreMesh(
        core_axis_name="c", subcore_axis_name="s",
        num_cores=_NC, num_subcores=_NS)

    @functools.partial(
        pl.kernel,
        out_type=[
            jax.ShapeDtypeStruct((_N, D), jnp.float32),
            jax.ShapeDtypeStruct((_N, D), jnp.float32),
            jax.ShapeDtypeStruct((_N,), jnp.float32),
            jax.ShapeDtypeStruct((_N,), jnp.float32),
        ],
        mesh=mesh,
        scratch_types=[
            pltpu.VMEM((_B,), jnp.int32),
            pltpu.VMEM((_B,), jnp.int32),
            pltpu.VMEM((_B,), jnp.float32),
            pltpu.VMEM((_B,), jnp.float32),
            pltpu.VMEM((_B,), jnp.float32),
            pltpu.VMEM((_B, D), jnp.float32),
            pltpu.VMEM((160, D), jnp.float32),
            pltpu.VMEM((1000,), jnp.float32),
            pltpu.VMEM((16,), jnp.float32),
            pltpu.VMEM_SHARED((_N, D), jnp.float32),
            pltpu.VMEM_SHARED((_N,), jnp.float32),
            pltpu.SemaphoreType.DMA,
        ],
    )
    def agg(h_hbm, es_hbm, ed_hbm, src_hbm, dst_hbm, c_hbm,
            num0_out, num1_out, den0_out, den1_out,
            src_v, dst_v, es_v, ed_v, w_v, rows_v, zrow_v, zden_v, c_v,
            num_s, den_s, sem):
        cid = lax.axis_index("c")
        sid = lax.axis_index("s")
        tid = cid * _NS + sid

        pltpu.sync_copy(c_hbm.at[pl.ds(0, 16)], c_v)

        zeros16 = jnp.zeros((16,), jnp.float32)

        def zrow_body(i, _):
            for j in range(D // 16):
                zrow_v[i, pl.ds(j * 16, 16)] = zeros16
            return 0
        lax.fori_loop(0, 160, zrow_body, 0)

        lo = pl.multiple_of(sid * 624, 8)

        @pl.when(sid < 15)
        def _():
            for k in range(3):
                pltpu.sync_copy(zrow_v, num_s.at[pl.ds(lo + k * 160, 160)])
            pltpu.sync_copy(zrow_v.at[pl.ds(0, 144)],
                            num_s.at[pl.ds(lo + 480, 144)])

        @pl.when(sid == 15)
        def _():
            for k in range(4):
                pltpu.sync_copy(zrow_v, num_s.at[pl.ds(9360 + k * 160, 160)])

        def zden_body(i, _):
            zden_v[pl.ds(i * 16, 16)] = zeros16
            return 0
        lax.fori_loop(0, 62, zden_body, 0)
        zden_v[pl.ds(984, 16)] = zeros16

        @pl.when(sid == 0)
        def _():
            for k in range(5):
                pltpu.sync_copy(zden_v, den_s.at[pl.ds(k * 1000, 1000)])

        @pl.when(sid == 8)
        def _():
            for k in range(5):
                pltpu.sync_copy(zden_v, den_s.at[pl.ds(5000 + k * 1000, 1000)])

        plsc.subcore_barrier()

        ebase = tid * _EPT

        def chunk(k, _):
            base = pl.multiple_of(ebase + k * _B, 8)
            pltpu.sync_copy(src_hbm.at[pl.ds(base, _B)], src_v)
            pltpu.sync_copy(dst_hbm.at[pl.ds(base, _B)], dst_v)
            pltpu.async_copy(es_hbm.at[src_v], es_v, sem).wait()
            pltpu.async_copy(ed_hbm.at[dst_v], ed_v, sem).wait()
            pltpu.async_copy(h_hbm.at[src_v], rows_v, sem).wait()

            cvec = c_v[...]
            for j in range(_B // 16):
                e = es_v[pl.ds(j * 16, 16)] + ed_v[pl.ds(j * 16, 16)]
                e = jnp.where(e >= 0.0, e, _SLOPE * e)
                w_v[pl.ds(j * 16, 16)] = jnp.exp(e - cvec)

            def sgrp(g, _):
                wvec = w_v[pl.ds(g * 16, 16)]
                for l in range(16):
                    wi = wvec[l]
                    i = g * 16 + l
                    for j in range(D // 16):
                        rows_v[i, pl.ds(j * 16, 16)] = (
                            rows_v[i, pl.ds(j * 16, 16)] * wi)
                return 0
            lax.fori_loop(0, _B // 16, sgrp, 0)

            pltpu.sync_copy(rows_v, num_s.at[dst_v], add=True)
            pltpu.sync_copy(w_v, den_s.at[dst_v], add=True)
            return 0
        lax.fori_loop(0, _CHUNKS, chunk, 0)

        plsc.subcore_barrier()

        def write_num(out_ref):
            @pl.when(sid < 15)
            def _():
                pltpu.sync_copy(num_s.at[pl.ds(lo, 624)],
                                out_ref.at[pl.ds(lo, 624)])

            @pl.when(sid == 15)
            def _():
                pltpu.sync_copy(num_s.at[pl.ds(9360, 640)],
                                out_ref.at[pl.ds(9360, 640)])

        @pl.when(cid == 0)
        def _():
            write_num(num0_out)

            @pl.when(sid == 0)
            def _():
                pltpu.sync_copy(den_s, den0_out)

        @pl.when(cid == 1)
        def _():
            write_num(num1_out)

            @pl.when(sid == 0)
            def _():
                pltpu.sync_copy(den_s, den1_out)

    return agg


_sc_agg_128 = _sc_aggregate(128)


def _dense1(x, W1, a1p):
    K, H = W1.shape

    def body(x_ref, w_ref, a_ref, h_ref, esd_ref, c_ref, m_ref):
        i = pl.program_id(0)
        h = lax.dot_general(x_ref[...], w_ref[...], (((1,), (0,)), ((), ())),
                            precision=_HIGH, preferred_element_type=jnp.float32)
        h_ref[...] = h
        esd = lax.dot_general(h, a_ref[...], (((1,), (1,)), ((), ())),
                              precision=_HIGH, preferred_element_type=jnp.float32)
        esd_ref[...] = esd
        m0 = jnp.max(esd[:, 0])
        m1 = jnp.max(esd[:, 1])

        @pl.when(i == 0)
        def _():
            m_ref[0] = m0
            m_ref[1] = m1

        @pl.when(i > 0)
        def _():
            m_ref[0] = jnp.maximum(m_ref[0], m0)
            m_ref[1] = jnp.maximum(m_ref[1], m1)

        s = m_ref[0] + m_ref[1]
        c = jnp.where(s >= 0.0, s, _SLOPE * s)
        c_ref[...] = jnp.full((1, 128), c)

    return pl.pallas_call(
        body,
        grid=(_N // _BLK,),
        in_specs=[
            pl.BlockSpec((_BLK, K), lambda i: (i, 0)),
            pl.BlockSpec((K, H), lambda i: (0, 0)),
            pl.BlockSpec((2, K), lambda i: (0, 0)),
        ],
        out_specs=[
            pl.BlockSpec((_BLK, H), lambda i: (i, 0)),
            pl.BlockSpec((_BLK, 2), lambda i: (i, 0)),
            pl.BlockSpec((1, 128), lambda i: (0, 0)),
        ],
        out_shape=[
            jax.ShapeDtypeStruct((_N, H), jnp.float32),
            jax.ShapeDtypeStruct((_N, 2), jnp.float32),
            jax.ShapeDtypeStruct((1, 128), jnp.float32),
        ],
        scratch_shapes=[pltpu.SMEM((2,), jnp.float32)],
    )(x, W1, a1p)


def _dense2(na, nb, da, db, W2, a2p):
    K, H = W2.shape

    def body(na_ref, nb_ref, da_ref, db_ref, w_ref, a_ref,
             g_ref, esd_ref, c_ref, m_ref):
        i = pl.program_id(0)
        den = da_ref[...] + db_ref[...]
        den = jnp.where(den == 0.0, 1.0, den)
        v = (na_ref[...] + nb_ref[...]) / den
        h2 = jnp.where(v > 0.0, v, jnp.exp(jnp.minimum(v, 0.0)) - 1.0)
        g = lax.dot_general(h2, w_ref[...], (((1,), (0,)), ((), ())),
                            precision=_HIGH, preferred_element_type=jnp.float32)
        g_ref[...] = jnp.concatenate(
            [g, jnp.zeros((_BLK, 128 - H), jnp.float32)], axis=1)
        esd = lax.dot_general(g, a_ref[...], (((1,), (1,)), ((), ())),
                              precision=_HIGH, preferred_element_type=jnp.float32)
        esd_ref[...] = esd
        m0 = jnp.max(esd[:, 0])
        m1 = jnp.max(esd[:, 1])

        @pl.when(i == 0)
        def _():
            m_ref[0] = m0
            m_ref[1] = m1

        @pl.when(i > 0)
        def _():
            m_ref[0] = jnp.maximum(m_ref[0], m0)
            m_ref[1] = jnp.maximum(m_ref[1], m1)

        s = m_ref[0] + m_ref[1]
        c = jnp.where(s >= 0.0, s, _SLOPE * s)
        c_ref[...] = jnp.full((1, 128), c)

    return pl.pallas_call(
        body,
        grid=(_N // _BLK,),
        in_specs=[
            pl.BlockSpec((_BLK, K), lambda i: (i, 0)),
            pl.BlockSpec((_BLK, K), lambda i: (i, 0)),
            pl.BlockSpec((_BLK, 1), lambda i: (i, 0)),
            pl.BlockSpec((_BLK, 1), lambda i: (i, 0)),
            pl.BlockSpec((K, H), lambda i: (0, 0)),
            pl.BlockSpec((2, H), lambda i: (0, 0)),
        ],
        out_specs=[
            pl.BlockSpec((_BLK, 128), lambda i: (i, 0)),
            pl.BlockSpec((_BLK, 2), lambda i: (i, 0)),
            pl.BlockSpec((1, 128), lambda i: (0, 0)),
        ],
        out_shape=[
            jax.ShapeDtypeStruct((_N, 128), jnp.float32),
            jax.ShapeDtypeStruct((_N, 2), jnp.float32),
            jax.ShapeDtypeStruct((1, 128), jnp.float32),
        ],
        scratch_shapes=[pltpu.SMEM((2,), jnp.float32)],
    )(na, nb, da, db, W2, a2p)


def _final(na, nb, da, db, H):
    def body(na_ref, nb_ref, da_ref, db_ref, o_ref):
        den = da_ref[...] + db_ref[...]
        den = jnp.where(den == 0.0, 1.0, den)
        o_ref[...] = (na_ref[..., :H] + nb_ref[..., :H]) / den

    return pl.pallas_call(
        body,
        grid=(_N // _BLK,),
        in_specs=[
            pl.BlockSpec((_BLK, 128), lambda i: (i, 0)),
            pl.BlockSpec((_BLK, 128), lambda i: (i, 0)),
            pl.BlockSpec((_BLK, 1), lambda i: (i, 0)),
            pl.BlockSpec((_BLK, 1), lambda i: (i, 0)),
        ],
        out_specs=pl.BlockSpec((_BLK, H), lambda i: (i, 0)),
        out_shape=jax.ShapeDtypeStruct((_N, H), jnp.float32),
    )(na, nb, da, db)


def kernel(x, adj, W1, a1_src, a1_dst, W2, a2_src, a2_dst):
    src = adj[0]
    dst = adj[1]
    a1p = jnp.stack([a1_src, a1_dst])
    a2p = jnp.stack([a2_src, a2_dst])

    h1, esd1, c1 = _dense1(x, W1, a1p)
    na1, nb1, da1, db1 = _sc_agg_128(h1, esd1[:, 0], esd1[:, 1], src, dst, c1[0])
    g2, esd2, c2 = _dense2(
        na1, nb1, da1.reshape(_N, 1), db1.reshape(_N, 1), W2, a2p)
    na2, nb2, da2, db2 = _sc_agg_128(
        g2, esd2[:, 0], esd2[:, 1], src, dst, c2[0])
    return _final(
        na2, nb2, da2.reshape(_N, 1), db2.reshape(_N, 1), W2.shape[1])


# speedup vs baseline: 47.4006x; 2.8944x over previous
import functools

import jax
import jax.numpy as jnp
from jax import lax
from jax.experimental import pallas as pl
from jax.experimental.pallas import tpu as pltpu
from jax.experimental.pallas import tpu_sc as plsc

_N = 10000
_E = 320000
_SLOPE = 0.2

_NC = 2
_NS = 16
_NT = _NC * _NS
_B = 80
_EPT = _E // _NT
_CHUNKS = _EPT // _B

_BLK = 1000
_HIGH = jax.lax.Precision.HIGHEST


def _sc_aggregate(D, d_valid):
    mesh = plsc.VectorSubcoreMesh(
        core_axis_name="c", subcore_axis_name="s",
        num_cores=_NC, num_subcores=_NS)
    ngd = d_valid // 16

    @functools.partial(
        pl.kernel,
        out_type=[
            jax.ShapeDtypeStruct((_N, D), jnp.float32),
            jax.ShapeDtypeStruct((_N, D), jnp.float32),
            jax.ShapeDtypeStruct((_N,), jnp.float32),
            jax.ShapeDtypeStruct((_N,), jnp.float32),
        ],
        mesh=mesh,
        scratch_types=(
            [pltpu.VMEM((_B, D), jnp.float32)] * 4 +
            [pltpu.VMEM((_B,), jnp.float32)] * 4 +
            [pltpu.VMEM((_B,), jnp.float32)] * 4 +
            [pltpu.VMEM((_B,), jnp.float32)] * 4 +
            [pltpu.VMEM((_B,), jnp.int32)] * 4 +
            [pltpu.VMEM((_B,), jnp.int32)] * 4 +
            [
                pltpu.VMEM((16,), jnp.float32),
                pltpu.VMEM((1000,), jnp.float32),
                pltpu.VMEM_SHARED((_N, D), jnp.float32),
                pltpu.VMEM_SHARED((_N,), jnp.float32),
            ] +
            [pltpu.SemaphoreType.DMA] * 12
        ),
    )
    def agg(h_hbm, es_hbm, ed_hbm, src_hbm, dst_hbm, c_hbm,
            num0_out, num1_out, den0_out, den1_out,
            *sc):
        rows = sc[0:4]
        esb = sc[4:8]
        edb = sc[8:12]
        wbuf = sc[12:16]
        six = sc[16:20]
        dix = sc[20:24]
        c_v, zden_v, num_s, den_s = sc[24:28]
        isem = sc[28:32]
        gsem = sc[32:36]
        ssem = sc[36:40]

        cid = lax.axis_index("c")
        sid = lax.axis_index("s")
        tid = cid * _NS + sid

        ebase = tid * _EPT
        pltpu.sync_copy(c_hbm.at[pl.ds(0, 16)], c_v)

        zeros16 = jnp.zeros((16,), jnp.float32)

        def zrow_body(i, _):
            for j in range(D // 16):
                rows[0][i, pl.ds(j * 16, 16)] = zeros16
            return 0
        lax.fori_loop(0, _B, zrow_body, 0)

        lo = pl.multiple_of(sid * 624, 8)

        @pl.when(sid < 15)
        def _():
            for k in range(7):
                pltpu.sync_copy(rows[0], num_s.at[pl.ds(lo + k * _B, _B)])
            pltpu.sync_copy(rows[0].at[pl.ds(0, 64)],
                            num_s.at[pl.ds(lo + 560, 64)])

        @pl.when(sid == 15)
        def _():
            for k in range(8):
                pltpu.sync_copy(rows[0], num_s.at[pl.ds(9360 + k * _B, _B)])

        def zden_body(i, _):
            zden_v[pl.ds(i * 16, 16)] = zeros16
            return 0
        lax.fori_loop(0, 62, zden_body, 0)
        zden_v[pl.ds(984, 16)] = zeros16

        @pl.when(sid == 0)
        def _():
            for k in range(5):
                pltpu.sync_copy(zden_v, den_s.at[pl.ds(k * 1000, 1000)])

        @pl.when(sid == 8)
        def _():
            for k in range(5):
                pltpu.sync_copy(zden_v, den_s.at[pl.ds(5000 + k * 1000, 1000)])

        plsc.subcore_barrier()

        def issue_idx(k, m):
            pltpu.async_copy(src_hbm.at[pl.ds(ebase + k * _B, _B)],
                             six[m], isem[m])
            pltpu.async_copy(dst_hbm.at[pl.ds(ebase + k * _B, _B)],
                             dix[m], isem[m])

        def wait_idx(k, m):
            pltpu.make_async_copy(src_hbm.at[pl.ds(ebase + k * _B, _B)],
                                  six[m], isem[m]).wait()
            pltpu.make_async_copy(dst_hbm.at[pl.ds(ebase + k * _B, _B)],
                                  dix[m], isem[m]).wait()

        def issue_gather(m):
            pltpu.async_copy(es_hbm.at[six[m]], esb[m], gsem[m])
            pltpu.async_copy(ed_hbm.at[dix[m]], edb[m], gsem[m])
            pltpu.async_copy(h_hbm.at[six[m]], rows[m], gsem[m])

        def wait_gather(m):
            pltpu.make_async_copy(es_hbm.at[six[m]], esb[m], gsem[m]).wait()
            pltpu.make_async_copy(ed_hbm.at[dix[m]], edb[m], gsem[m]).wait()
            pltpu.make_async_copy(h_hbm.at[six[m]], rows[m], gsem[m]).wait()

        def issue_scatter(m):
            pltpu.async_copy(rows[m], num_s.at[dix[m]], ssem[m], add=True)
            pltpu.async_copy(wbuf[m], den_s.at[dix[m]], ssem[m], add=True)

        def wait_scatter(m):
            pltpu.make_async_copy(rows[m], num_s.at[dix[m]], ssem[m]).wait()
            pltpu.make_async_copy(wbuf[m], den_s.at[dix[m]], ssem[m]).wait()

        def compute_w(m):
            cvec = c_v[...]
            for g in range(_B // 16):
                es16 = esb[m][pl.ds(g * 16, 16)]
                ed16 = edb[m][pl.ds(g * 16, 16)]
                e = es16 + ed16
                e = jnp.where(e >= 0.0, e, _SLOPE * e)
                wbuf[m][pl.ds(g * 16, 16)] = jnp.exp(e - cvec)

        def scale_rows(m):
            rb = rows[m]
            wb = wbuf[m]

            def sgrp(g, _):
                wvec = wb[pl.ds(g * 16, 16)]
                for l in range(16):
                    wi = wvec[l]
                    i = g * 16 + l
                    for j in range(ngd):
                        rb[i, pl.ds(j * 16, 16)] = (
                            rb[i, pl.ds(j * 16, 16)] * wi)
                return 0
            lax.fori_loop(0, _B // 16, sgrp, 0)

        def body(m):
            wait_gather(m)
            compute_w(m)
            scale_rows(m)
            issue_scatter(m)

        issue_idx(0, 0)
        issue_idx(1, 1)
        wait_idx(0, 0)
        issue_gather(0)
        issue_idx(2, 2)
        wait_idx(1, 1)
        issue_gather(1)
        issue_idx(3, 3)
        body(0)
        wait_idx(2, 2)
        issue_gather(2)
        body(1)

        def quad(i, _):
            def slot(k, m):
                wait_scatter(m)
                issue_idx(k + 2, m)
                mg = (m + 3) % 4
                wait_idx(k + 1, mg)
                issue_gather(mg)
                body((m + 2) % 4)

            k0 = 4 * i + 2
            slot(k0, 0)
            slot(k0 + 1, 1)
            slot(k0 + 2, 2)
            slot(k0 + 3, 3)
            return 0
        lax.fori_loop(0, 30, quad, 0)

        wait_scatter(0)
        issue_idx(124, 0)
        wait_idx(123, 3)
        issue_gather(3)
        body(2)
        wait_scatter(1)
        wait_idx(124, 0)
        issue_gather(0)
        body(3)
        wait_scatter(2)
        body(0)
        wait_scatter(3)
        wait_scatter(0)

        plsc.subcore_barrier()

        def write_num(out_ref):
            @pl.when(sid < 15)
            def _():
                pltpu.sync_copy(num_s.at[pl.ds(lo, 624)],
                                out_ref.at[pl.ds(lo, 624)])

            @pl.when(sid == 15)
            def _():
                pltpu.sync_copy(num_s.at[pl.ds(9360, 640)],
                                out_ref.at[pl.ds(9360, 640)])

        @pl.when(cid == 0)
        def _():
            write_num(num0_out)

            @pl.when(sid == 0)
            def _():
                pltpu.sync_copy(den_s, den0_out)

        @pl.when(cid == 1)
        def _():
            write_num(num1_out)

            @pl.when(sid == 0)
            def _():
                pltpu.sync_copy(den_s, den1_out)

    return agg


_sc_agg_l1 = _sc_aggregate(128, 128)
_sc_agg_l2 = _sc_aggregate(128, 64)


def _dense1(x, W1, a1p):
    K, H = W1.shape

    def body(x_ref, w_ref, a_ref, h_ref, esd_ref, c_ref, m_ref):
        i = pl.program_id(0)
        h = lax.dot_general(x_ref[...], w_ref[...], (((1,), (0,)), ((), ())),
                            precision=_HIGH, preferred_element_type=jnp.float32)
        h_ref[...] = h
        esd = lax.dot_general(h, a_ref[...], (((1,), (1,)), ((), ())),
                              precision=_HIGH, preferred_element_type=jnp.float32)
        esd_ref[...] = esd
        m0 = jnp.max(esd[:, 0])
        m1 = jnp.max(esd[:, 1])

        @pl.when(i == 0)
        def _():
            m_ref[0] = m0
            m_ref[1] = m1

        @pl.when(i > 0)
        def _():
            m_ref[0] = jnp.maximum(m_ref[0], m0)
            m_ref[1] = jnp.maximum(m_ref[1], m1)

        s = m_ref[0] + m_ref[1]
        c = jnp.where(s >= 0.0, s, _SLOPE * s)
        c_ref[...] = jnp.full((1, 128), c)

    return pl.pallas_call(
        body,
        grid=(_N // _BLK,),
        in_specs=[
            pl.BlockSpec((_BLK, K), lambda i: (i, 0)),
            pl.BlockSpec((K, H), lambda i: (0, 0)),
            pl.BlockSpec((2, K), lambda i: (0, 0)),
        ],
        out_specs=[
            pl.BlockSpec((_BLK, H), lambda i: (i, 0)),
            pl.BlockSpec((_BLK, 2), lambda i: (i, 0)),
            pl.BlockSpec((1, 128), lambda i: (0, 0)),
        ],
        out_shape=[
            jax.ShapeDtypeStruct((_N, H), jnp.float32),
            jax.ShapeDtypeStruct((_N, 2), jnp.float32),
            jax.ShapeDtypeStruct((1, 128), jnp.float32),
        ],
        scratch_shapes=[pltpu.SMEM((2,), jnp.float32)],
    )(x, W1, a1p)


def _dense2(na, nb, da, db, W2, a2p):
    K, H = W2.shape

    def body(na_ref, nb_ref, da_ref, db_ref, w_ref, a_ref,
             g_ref, esd_ref, c_ref, m_ref):
        i = pl.program_id(0)
        den = da_ref[...] + db_ref[...]
        den = jnp.where(den == 0.0, 1.0, den)
        v = (na_ref[...] + nb_ref[...]) / den
        h2 = jnp.where(v > 0.0, v, jnp.exp(jnp.minimum(v, 0.0)) - 1.0)
        g = lax.dot_general(h2, w_ref[...], (((1,), (0,)), ((), ())),
                            precision=_HIGH, preferred_element_type=jnp.float32)
        g_ref[...] = jnp.concatenate(
            [g, jnp.zeros((_BLK, 128 - H), jnp.float32)], axis=1)
        esd = lax.dot_general(g, a_ref[...], (((1,), (1,)), ((), ())),
                              precision=_HIGH, preferred_element_type=jnp.float32)
        esd_ref[...] = esd
        m0 = jnp.max(esd[:, 0])
        m1 = jnp.max(esd[:, 1])

        @pl.when(i == 0)
        def _():
            m_ref[0] = m0
            m_ref[1] = m1

        @pl.when(i > 0)
        def _():
            m_ref[0] = jnp.maximum(m_ref[0], m0)
            m_ref[1] = jnp.maximum(m_ref[1], m1)

        s = m_ref[0] + m_ref[1]
        c = jnp.where(s >= 0.0, s, _SLOPE * s)
        c_ref[...] = jnp.full((1, 128), c)

    return pl.pallas_call(
        body,
        grid=(_N // _BLK,),
        in_specs=[
            pl.BlockSpec((_BLK, K), lambda i: (i, 0)),
            pl.BlockSpec((_BLK, K), lambda i: (i, 0)),
            pl.BlockSpec((_BLK, 1), lambda i: (i, 0)),
            pl.BlockSpec((_BLK, 1), lambda i: (i, 0)),
            pl.BlockSpec((K, H), lambda i: (0, 0)),
            pl.BlockSpec((2, H), lambda i: (0, 0)),
        ],
        out_specs=[
            pl.BlockSpec((_BLK, 128), lambda i: (i, 0)),
            pl.BlockSpec((_BLK, 2), lambda i: (i, 0)),
            pl.BlockSpec((1, 128), lambda i: (0, 0)),
        ],
        out_shape=[
            jax.ShapeDtypeStruct((_N, 128), jnp.float32),
            jax.ShapeDtypeStruct((_N, 2), jnp.float32),
            jax.ShapeDtypeStruct((1, 128), jnp.float32),
        ],
        scratch_shapes=[pltpu.SMEM((2,), jnp.float32)],
    )(na, nb, da, db, W2, a2p)


def _final(na, nb, da, db, H):
    def body(na_ref, nb_ref, da_ref, db_ref, o_ref):
        den = da_ref[...] + db_ref[...]
        den = jnp.where(den == 0.0, 1.0, den)
        o_ref[...] = (na_ref[..., :H] + nb_ref[..., :H]) / den

    return pl.pallas_call(
        body,
        grid=(_N // _BLK,),
        in_specs=[
            pl.BlockSpec((_BLK, 128), lambda i: (i, 0)),
            pl.BlockSpec((_BLK, 128), lambda i: (i, 0)),
            pl.BlockSpec((_BLK, 1), lambda i: (i, 0)),
            pl.BlockSpec((_BLK, 1), lambda i: (i, 0)),
        ],
        out_specs=pl.BlockSpec((_BLK, H), lambda i: (i, 0)),
        out_shape=jax.ShapeDtypeStruct((_N, H), jnp.float32),
    )(na, nb, da, db)


def kernel(x, adj, W1, a1_src, a1_dst, W2, a2_src, a2_dst):
    src = adj[0]
    dst = adj[1]
    a1p = jnp.stack([a1_src, a1_dst])
    a2p = jnp.stack([a2_src, a2_dst])

    h1, esd1, c1 = _dense1(x, W1, a1p)
    na1, nb1, da1, db1 = _sc_agg_l1(h1, esd1[:, 0], esd1[:, 1], src, dst, c1[0])
    g2, esd2, c2 = _dense2(
        na1, nb1, da1.reshape(_N, 1), db1.reshape(_N, 1), W2, a2p)
    na2, nb2, da2, db2 = _sc_agg_l2(
        g2, esd2[:, 0], esd2[:, 1], src, dst, c2[0])
    return _final(
        na2, nb2, da2.reshape(_N, 1), db2.reshape(_N, 1), W2.shape[1])


# speedup vs baseline: 52.3988x; 1.1054x over previous
import functools

import jax
import jax.numpy as jnp
from jax import lax
from jax.experimental import pallas as pl
from jax.experimental.pallas import tpu as pltpu
from jax.experimental.pallas import tpu_sc as plsc

_N = 10000
_E = 320000
_SLOPE = 0.2

_NC = 2
_NS = 16
_NT = _NC * _NS
_B = 80
_EPT = _E // _NT
_CHUNKS = _EPT // _B

_BLK = 2000
_HIGH = jax.lax.Precision.HIGHEST


def _sc_aggregate(D, d_valid):
    mesh = plsc.VectorSubcoreMesh(
        core_axis_name="c", subcore_axis_name="s",
        num_cores=_NC, num_subcores=_NS)
    ngd = d_valid // 16

    @functools.partial(
        pl.kernel,
        out_type=[
            jax.ShapeDtypeStruct((_N, D), jnp.float32),
            jax.ShapeDtypeStruct((_N, D), jnp.float32),
            jax.ShapeDtypeStruct((_N,), jnp.float32),
            jax.ShapeDtypeStruct((_N,), jnp.float32),
        ],
        mesh=mesh,
        scratch_types=(
            [pltpu.VMEM((_B, D), jnp.float32)] * 4 +
            [pltpu.VMEM((_B,), jnp.float32)] * 4 +
            [pltpu.VMEM((_B,), jnp.float32)] * 4 +
            [pltpu.VMEM((_B,), jnp.float32)] * 4 +
            [pltpu.VMEM((_B,), jnp.int32)] * 4 +
            [pltpu.VMEM((_B,), jnp.int32)] * 4 +
            [
                pltpu.VMEM((16,), jnp.float32),
                pltpu.VMEM((1000,), jnp.float32),
                pltpu.VMEM_SHARED((_N, D), jnp.float32),
                pltpu.VMEM_SHARED((_N,), jnp.float32),
            ] +
            [pltpu.SemaphoreType.DMA] * 12
        ),
    )
    def agg(h_hbm, es_hbm, ed_hbm, src_hbm, dst_hbm, c_hbm,
            num0_out, num1_out, den0_out, den1_out,
            *sc):
        rows = sc[0:4]
        esb = sc[4:8]
        edb = sc[8:12]
        wbuf = sc[12:16]
        six = sc[16:20]
        dix = sc[20:24]
        c_v, zden_v, num_s, den_s = sc[24:28]
        isem = sc[28:32]
        gsem = sc[32:36]
        ssem = sc[36:40]

        cid = lax.axis_index("c")
        sid = lax.axis_index("s")
        tid = cid * _NS + sid

        ebase = tid * _EPT
        pltpu.sync_copy(c_hbm.at[0, pl.ds(0, 16)], c_v)

        zeros16 = jnp.zeros((16,), jnp.float32)

        def zrow_body(i, _):
            for j in range(D // 16):
                rows[0][i, pl.ds(j * 16, 16)] = zeros16
            return 0
        lax.fori_loop(0, _B, zrow_body, 0)

        lo = pl.multiple_of(sid * 624, 8)

        @pl.when(sid < 15)
        def _():
            for k in range(7):
                pltpu.sync_copy(rows[0], num_s.at[pl.ds(lo + k * _B, _B)])
            pltpu.sync_copy(rows[0].at[pl.ds(0, 64)],
                            num_s.at[pl.ds(lo + 560, 64)])

        @pl.when(sid == 15)
        def _():
            for k in range(8):
                pltpu.sync_copy(rows[0], num_s.at[pl.ds(9360 + k * _B, _B)])

        if d_valid == D:
            def zden_body(i, _):
                zden_v[pl.ds(i * 16, 16)] = zeros16
                return 0
            lax.fori_loop(0, 62, zden_body, 0)
            zden_v[pl.ds(984, 16)] = zeros16

            @pl.when(sid == 0)
            def _():
                for k in range(5):
                    pltpu.sync_copy(zden_v, den_s.at[pl.ds(k * 1000, 1000)])

            @pl.when(sid == 8)
            def _():
                for k in range(5):
                    pltpu.sync_copy(zden_v,
                                    den_s.at[pl.ds(5000 + k * 1000, 1000)])

        plsc.subcore_barrier()

        def issue_idx(k, m):
            pltpu.async_copy(src_hbm.at[pl.ds(ebase + k * _B, _B)],
                             six[m], isem[m])
            pltpu.async_copy(dst_hbm.at[pl.ds(ebase + k * _B, _B)],
                             dix[m], isem[m])

        def wait_idx(k, m):
            pltpu.make_async_copy(src_hbm.at[pl.ds(ebase + k * _B, _B)],
                                  six[m], isem[m]).wait()
            pltpu.make_async_copy(dst_hbm.at[pl.ds(ebase + k * _B, _B)],
                                  dix[m], isem[m]).wait()

        def issue_gather(m):
            pltpu.async_copy(es_hbm.at[six[m]], esb[m], gsem[m])
            pltpu.async_copy(ed_hbm.at[dix[m]], edb[m], gsem[m])
            pltpu.async_copy(h_hbm.at[six[m]], rows[m], gsem[m])

        def wait_gather(m):
            pltpu.make_async_copy(es_hbm.at[six[m]], esb[m], gsem[m]).wait()
            pltpu.make_async_copy(ed_hbm.at[dix[m]], edb[m], gsem[m]).wait()
            pltpu.make_async_copy(h_hbm.at[six[m]], rows[m], gsem[m]).wait()

        def issue_scatter(m):
            pltpu.async_copy(rows[m], num_s.at[dix[m]], ssem[m], add=True)
            if d_valid == D:
                pltpu.async_copy(wbuf[m], den_s.at[dix[m]], ssem[m], add=True)

        def wait_scatter(m):
            pltpu.make_async_copy(rows[m], num_s.at[dix[m]], ssem[m]).wait()
            if d_valid == D:
                pltpu.make_async_copy(
                    wbuf[m], den_s.at[dix[m]], ssem[m]).wait()

        def compute_w(m):
            cvec = c_v[...]
            for g in range(_B // 16):
                es16 = esb[m][pl.ds(g * 16, 16)]
                ed16 = edb[m][pl.ds(g * 16, 16)]
                e = es16 + ed16
                e = jnp.where(e >= 0.0, e, _SLOPE * e)
                wbuf[m][pl.ds(g * 16, 16)] = jnp.exp(e - cvec)

        def scale_rows(m):
            rb = rows[m]
            wb = wbuf[m]

            def sgrp(g, _):
                wvec = wb[pl.ds(g * 16, 16)]
                for l in range(16):
                    wi = wvec[l]
                    i = g * 16 + l
                    for j in range(ngd):
                        rb[i, pl.ds(j * 16, 16)] = (
                            rb[i, pl.ds(j * 16, 16)] * wi)
                    if d_valid < D:
                        rb[i, pl.ds(d_valid, 16)] = jnp.full(
                            (16,), wi, jnp.float32)
                return 0
            lax.fori_loop(0, _B // 16, sgrp, 0)

        def body(m):
            wait_gather(m)
            compute_w(m)
            scale_rows(m)
            issue_scatter(m)

        issue_idx(0, 0)
        issue_idx(1, 1)
        wait_idx(0, 0)
        issue_gather(0)
        issue_idx(2, 2)
        wait_idx(1, 1)
        issue_gather(1)
        issue_idx(3, 3)
        body(0)
        wait_idx(2, 2)
        issue_gather(2)
        body(1)

        def quad(i, _):
            def slot(k, m):
                wait_scatter(m)
                issue_idx(k + 2, m)
                mg = (m + 3) % 4
                wait_idx(k + 1, mg)
                issue_gather(mg)
                body((m + 2) % 4)

            k0 = 4 * i + 2
            slot(k0, 0)
            slot(k0 + 1, 1)
            slot(k0 + 2, 2)
            slot(k0 + 3, 3)
            return 0
        lax.fori_loop(0, 30, quad, 0)

        wait_scatter(0)
        issue_idx(124, 0)
        wait_idx(123, 3)
        issue_gather(3)
        body(2)
        wait_scatter(1)
        wait_idx(124, 0)
        issue_gather(0)
        body(3)
        wait_scatter(2)
        body(0)
        wait_scatter(3)
        wait_scatter(0)

        plsc.subcore_barrier()

        def write_num(out_ref):
            @pl.when(sid < 15)
            def _():
                pltpu.sync_copy(num_s.at[pl.ds(lo, 624)],
                                out_ref.at[pl.ds(lo, 624)])

            @pl.when(sid == 15)
            def _():
                pltpu.sync_copy(num_s.at[pl.ds(9360, 640)],
                                out_ref.at[pl.ds(9360, 640)])

        @pl.when(cid == 0)
        def _():
            write_num(num0_out)

            if d_valid == D:
                @pl.when(sid == 0)
                def _():
                    pltpu.sync_copy(den_s, den0_out)

        @pl.when(cid == 1)
        def _():
            write_num(num1_out)

            if d_valid == D:
                @pl.when(sid == 0)
                def _():
                    pltpu.sync_copy(den_s, den1_out)

    return agg


_sc_agg_l1 = _sc_aggregate(128, 128)
_sc_agg_l2 = _sc_aggregate(128, 64)


def _dense1(x, W1, a1p):
    K, H = W1.shape

    def body(x_ref, w_ref, a_ref, h_ref, esd_ref, c_ref, m_ref):
        i = pl.program_id(0)
        h = lax.dot_general(x_ref[...], w_ref[...], (((1,), (0,)), ((), ())),
                            precision=_HIGH, preferred_element_type=jnp.float32)
        h_ref[...] = h
        esd = lax.dot_general(h, a_ref[...], (((1,), (1,)), ((), ())),
                              precision=_HIGH, preferred_element_type=jnp.float32)
        esd_ref[...] = esd
        m0 = jnp.max(esd[:, 0])
        m1 = jnp.max(esd[:, 1])

        @pl.when(i == 0)
        def _():
            m_ref[0] = m0
            m_ref[1] = m1

        @pl.when(i > 0)
        def _():
            m_ref[0] = jnp.maximum(m_ref[0], m0)
            m_ref[1] = jnp.maximum(m_ref[1], m1)

        s = m_ref[0] + m_ref[1]
        c = jnp.where(s >= 0.0, s, _SLOPE * s)
        c_ref[...] = jnp.full((1, 128), c)

    return pl.pallas_call(
        body,
        grid=(_N // _BLK,),
        in_specs=[
            pl.BlockSpec((_BLK, K), lambda i: (i, 0)),
            pl.BlockSpec((K, H), lambda i: (0, 0)),
            pl.BlockSpec((2, K), lambda i: (0, 0)),
        ],
        out_specs=[
            pl.BlockSpec((_BLK, H), lambda i: (i, 0)),
            pl.BlockSpec((_BLK, 2), lambda i: (i, 0)),
            pl.BlockSpec((1, 128), lambda i: (0, 0)),
        ],
        out_shape=[
            jax.ShapeDtypeStruct((_N, H), jnp.float32),
            jax.ShapeDtypeStruct((_N, 2), jnp.float32),
            jax.ShapeDtypeStruct((1, 128), jnp.float32),
        ],
        scratch_shapes=[pltpu.SMEM((2,), jnp.float32)],
    )(x, W1, a1p)


def _dense2(na, nb, da, db, W2, a2p):
    K, H = W2.shape

    def body(na_ref, nb_ref, da_ref, db_ref, w_ref, a_ref,
             g_ref, esd_ref, c_ref, m_ref):
        i = pl.program_id(0)
        den = da_ref[...] + db_ref[...]
        den = jnp.where(den == 0.0, 1.0, den)
        v = (na_ref[...] + nb_ref[...]) / den
        h2 = jnp.where(v > 0.0, v, jnp.exp(jnp.minimum(v, 0.0)) - 1.0)
        g = lax.dot_general(h2, w_ref[...], (((1,), (0,)), ((), ())),
                            precision=_HIGH, preferred_element_type=jnp.float32)
        g_ref[...] = jnp.concatenate(
            [g, jnp.zeros((_BLK, 128 - H), jnp.float32)], axis=1)
        esd = lax.dot_general(g, a_ref[...], (((1,), (1,)), ((), ())),
                              precision=_HIGH, preferred_element_type=jnp.float32)
        esd_ref[...] = esd
        m0 = jnp.max(esd[:, 0])
        m1 = jnp.max(esd[:, 1])

        @pl.when(i == 0)
        def _():
            m_ref[0] = m0
            m_ref[1] = m1

        @pl.when(i > 0)
        def _():
            m_ref[0] = jnp.maximum(m_ref[0], m0)
            m_ref[1] = jnp.maximum(m_ref[1], m1)

        s = m_ref[0] + m_ref[1]
        c = jnp.where(s >= 0.0, s, _SLOPE * s)
        c_ref[...] = jnp.full((1, 128), c)

    return pl.pallas_call(
        body,
        grid=(_N // _BLK,),
        in_specs=[
            pl.BlockSpec((_BLK, K), lambda i: (i, 0)),
            pl.BlockSpec((_BLK, K), lambda i: (i, 0)),
            pl.BlockSpec((_BLK, 1), lambda i: (i, 0)),
            pl.BlockSpec((_BLK, 1), lambda i: (i, 0)),
            pl.BlockSpec((K, H), lambda i: (0, 0)),
            pl.BlockSpec((2, H), lambda i: (0, 0)),
        ],
        out_specs=[
            pl.BlockSpec((_BLK, 128), lambda i: (i, 0)),
            pl.BlockSpec((_BLK, 2), lambda i: (i, 0)),
            pl.BlockSpec((1, 128), lambda i: (0, 0)),
        ],
        out_shape=[
            jax.ShapeDtypeStruct((_N, 128), jnp.float32),
            jax.ShapeDtypeStruct((_N, 2), jnp.float32),
            jax.ShapeDtypeStruct((1, 128), jnp.float32),
        ],
        scratch_shapes=[pltpu.SMEM((2,), jnp.float32)],
    )(na, nb, da, db, W2, a2p)


def _final(na, nb, H):
    def body(na_ref, nb_ref, o_ref):
        den = na_ref[..., H:H + 1] + nb_ref[..., H:H + 1]
        den = jnp.where(den == 0.0, 1.0, den)
        o_ref[...] = (na_ref[..., :H] + nb_ref[..., :H]) / den

    return pl.pallas_call(
        body,
        grid=(_N // _BLK,),
        in_specs=[
            pl.BlockSpec((_BLK, 128), lambda i: (i, 0)),
            pl.BlockSpec((_BLK, 128), lambda i: (i, 0)),
        ],
        out_specs=pl.BlockSpec((_BLK, H), lambda i: (i, 0)),
        out_shape=jax.ShapeDtypeStruct((_N, H), jnp.float32),
    )(na, nb)


def kernel(x, adj, W1, a1_src, a1_dst, W2, a2_src, a2_dst):
    a1p = jnp.stack([a1_src, a1_dst])
    a2p = jnp.stack([a2_src, a2_dst])

    h1, esd1, c1 = _dense1(x, W1, a1p)
    src = adj[0]
    dst = adj[1]
    na1, nb1, da1, db1 = _sc_agg_l1(h1, esd1[:, 0], esd1[:, 1], src, dst, c1)
    g2, esd2, c2 = _dense2(
        na1, nb1, da1.reshape(_N, 1), db1.reshape(_N, 1), W2, a2p)
    na2, nb2, _, _ = _sc_agg_l2(g2, esd2[:, 0], esd2[:, 1], src, dst, c2)
    return _final(na2, nb2, W2.shape[1])
